# Initial kernel scaffold; baseline (speedup 1.0000x reference)
import jax
import jax.numpy as jnp
from jax.experimental import pallas as pl


def kernel(h, edge_row, edge_col, A_val, Wq, bq, Wk, bk, Wv, bv):
    raise NotImplementedError("write your pallas kernel here")


# speedup vs baseline: 41.8074x; 41.8074x over previous
import functools

import jax
import jax.numpy as jnp
from jax import lax
from jax.experimental import pallas as pl
from jax.experimental.pallas import tpu as pltpu
from jax.experimental.pallas import tpu_sc as plsc

_N = 10000
_E = 320000
_HID = 128
_NH = 8
_HD = _HID // _NH
_SCALE = _HD ** (-0.5)

_NC = 2
_NS = 16
_L = 16
_NW = _NC * _NS
_RPW = 320
_HALVES = 2
_RPH = _RPW // _HALVES
_NPAD = _NW * _RPW
_NSEG = _NW * _HALVES
_CHUNK = 32
_SUPER = 256
_CPS = _SUPER // _CHUNK
_NV = _HID // _L
_WPTR_PAD = 96
_NEG = -1e30


def _lane_gather(x, idx):
    dn = lax.GatherDimensionNumbers(offset_dims=(), collapsed_slice_dims=(0,),
                                    start_index_map=(0,))
    return lax.gather(x, idx[:, None], dn, slice_sizes=(1,),
                      mode=lax.GatherScatterMode.PROMISE_IN_BOUNDS)


def _proj_body(h_ref, wq, bq, wk, bk, wv, bv, q_ref, k_ref, v_ref):
    hb = h_ref[...]
    q_ref[...] = (jnp.dot(hb, wq[...], preferred_element_type=jnp.float32)
                  + bq[...]) * _SCALE
    k_ref[...] = jnp.dot(hb, wk[...], preferred_element_type=jnp.float32) + bk[...]
    v_ref[...] = jnp.dot(hb, wv[...], preferred_element_type=jnp.float32) + bv[...]


def _projections(hpad, Wq, bq, Wk, bk, Wv, bv):
    blk = 2560
    grid = (_NPAD // blk,)
    bs_h = pl.BlockSpec((blk, _HID), lambda i: (i, 0))
    bs_w = pl.BlockSpec((_HID, _HID), lambda i: (0, 0))
    bs_b = pl.BlockSpec((1, _HID), lambda i: (0, 0))
    out_sd = jax.ShapeDtypeStruct((_NPAD, _HID), jnp.float32)
    return pl.pallas_call(
        _proj_body,
        grid=grid,
        in_specs=[bs_h, bs_w, bs_b, bs_w, bs_b, bs_w, bs_b],
        out_specs=[bs_h, bs_h, bs_h],
        out_shape=[out_sd, out_sd, out_sd],
    )(hpad, Wq, bq.reshape(1, _HID), Wk, bk.reshape(1, _HID),
      Wv, bv.reshape(1, _HID))


_mesh = plsc.VectorSubcoreMesh(core_axis_name="c", subcore_axis_name="s",
                               num_cores=_NC, num_subcores=_NS)


@functools.partial(
    pl.kernel,
    out_type=jax.ShapeDtypeStruct((_NPAD, _HID), jnp.float32),
    mesh=_mesh,
    scratch_types=[
        pltpu.VMEM((_RPH + 1, _HID), jnp.float32),
        pltpu.VMEM((_RPH + 1, _HID), jnp.float32),
        pltpu.VMEM((_RPH + 1, _L), jnp.float32),
        pltpu.VMEM((_SUPER,), jnp.int32),
        pltpu.VMEM((_SUPER,), jnp.int32),
        pltpu.VMEM((_SUPER,), jnp.float32),
        pltpu.VMEM((_CHUNK, _HID), jnp.float32),
        pltpu.VMEM((_CHUNK, _HID), jnp.float32),
        pltpu.VMEM((_WPTR_PAD,), jnp.int32),
        pltpu.SemaphoreType.DMA,
        pltpu.SemaphoreType.DMA,
    ],
)
def _sc_attn(q_hbm, k_hbm, v_hbm, ecol_hbm, erow_hbm, aval_hbm, wptr_hbm,
             out_hbm, q_v, out_v, den_v, col_v, row_v, aval_v, k_v, v_v,
             wptr_v, sem_k, sem_v):
    wid = lax.axis_index("s") * _NC + lax.axis_index("c")
    pltpu.sync_copy(wptr_hbm, wptr_v)

    lanes = lax.iota(jnp.int32, _L)
    idx_lo = lanes % _NH
    idx_hi = idx_lo + _NH
    zf = jnp.zeros((_L,), jnp.float32)
    of = jnp.ones((_L,), jnp.float32)
    negf = jnp.full((_L,), _NEG, jnp.float32)

    def half_body(hh, carry):
        seg = wid * _HALVES + hh
        row0 = seg * _RPH
        pltpu.sync_copy(q_hbm.at[pl.ds(row0, _RPH)], q_v.at[pl.ds(0, _RPH)])
        e_start = wptr_v[pl.ds(seg, _L)][0]
        e_end = wptr_v[pl.ds(seg + 1, _L)][0]
        c_lo = e_start // _CHUNK
        c_hi = (e_end + _CHUNK - 1) // _CHUNK
        s_lo = e_start // _SUPER
        s_hi = (e_end + _SUPER - 1) // _SUPER

        def initrow(r, c):
            for j in range(_NV):
                out_v[r, pl.ds(j * _L, _L)] = zf
            den_v[r, pl.ds(0, _L)] = of
            return c

        lax.fori_loop(0, _RPH, initrow, 0)

        def edge_i(scb, off, i, rvec, avec, state):
            ein = scb + off + i
            valid = jnp.logical_and(ein >= e_start, ein < e_end)
            m, lsum, acc, prev = state
            r = rvec[i % _L]
            rl = jnp.where(valid, r - row0, _RPH)
            s = q_v[rl, pl.ds(0, _L)] * k_v[i, pl.ds(0, _L)]
            for j in range(1, _NV):
                s = s + q_v[rl, pl.ds(j * _L, _L)] * k_v[i, pl.ds(j * _L, _L)]
            g0 = _lane_gather(s, idx_lo)
            g1 = _lane_gather(s, idx_hi)
            a = avec[i % _L]
            sd = jnp.where(valid, (g0 + g1) * a, negf)
            is_new = jnp.logical_and(valid, r != prev)
            m_eff = jnp.where(is_new, negf, m)
            m_new = jnp.maximum(m_eff, sd)
            corr = jnp.exp(m_eff - m_new)
            p = jnp.exp(sd - m_new)
            lsum_n = lsum * corr + p
            acc_n = tuple(acc[j] * corr + p * v_v[i, pl.ds(j * _L, _L)]
                          for j in range(_NV))
            for j in range(_NV):
                out_v[rl, pl.ds(j * _L, _L)] = acc_n[j]
            den_v[rl, pl.ds(0, _L)] = lsum_n
            prev_n = jnp.where(valid, r, prev)
            return (m_new, lsum_n, acc_n, prev_n)

        def chunk_body(scb, cc, state):
            off = cc * _CHUNK
            idx_ref = col_v.at[pl.ds(off, _CHUNK)]
            cp_k = pltpu.async_copy(k_hbm.at[idx_ref], k_v, sem_k)
            cp_v = pltpu.async_copy(v_hbm.at[idx_ref], v_v, sem_v)
            cp_k.wait()
            cp_v.wait()
            rvs = [row_v[pl.ds(off + g * _L, _L)] for g in range(_CHUNK // _L)]
            avs = [aval_v[pl.ds(off + g * _L, _L)] for g in range(_CHUNK // _L)]
            for i in range(_CHUNK):
                state = edge_i(scb, off, i, rvs[i // _L], avs[i // _L], state)
            return state

        def super_body(sc_i, state):
            scb = sc_i * _SUPER
            pltpu.sync_copy(ecol_hbm.at[pl.ds(scb, _SUPER)], col_v)
            pltpu.sync_copy(erow_hbm.at[pl.ds(scb, _SUPER)], row_v)
            pltpu.sync_copy(aval_hbm.at[pl.ds(scb, _SUPER)], aval_v)
            lo = jnp.maximum(0, c_lo - sc_i * _CPS)
            hi = jnp.minimum(_CPS, c_hi - sc_i * _CPS)
            return lax.fori_loop(lo, hi,
                                 lambda cc, st: chunk_body(scb, cc, st), state)

        init_state = (negf, zf, tuple(zf for _ in range(_NV)), jnp.int32(-1))
        lax.fori_loop(s_lo, s_hi, super_body, init_state)

        def fin_row(r, c):
            d = den_v[r, pl.ds(0, _L)]
            for j in range(_NV):
                out_v[r, pl.ds(j * _L, _L)] = out_v[r, pl.ds(j * _L, _L)] / d
            return c

        lax.fori_loop(0, _RPH, fin_row, 0)
        pltpu.sync_copy(out_v.at[pl.ds(0, _RPH)],
                        out_hbm.at[pl.ds(row0, _RPH)])
        return carry

    lax.fori_loop(0, _HALVES, half_body, 0)


def kernel(h, edge_row, edge_col, A_val, Wq, bq, Wk, bk, Wv, bv):
    hpad = jnp.pad(h, ((0, _NPAD - _N), (0, 0)))
    q, k, v = _projections(hpad, Wq, bq, Wk, bk, Wv, bv)
    erow = edge_row.astype(jnp.int32)
    ecol = edge_col.astype(jnp.int32)
    bounds = (jnp.arange(_NSEG + 1, dtype=jnp.int32) * _RPH).astype(erow.dtype)
    wptr = jnp.searchsorted(erow, bounds, side="left").astype(jnp.int32)
    wptr_pad = jnp.concatenate(
        [wptr, jnp.full((_WPTR_PAD - _NSEG - 1,), _E, jnp.int32)])
    out = _sc_attn(q, k, v, ecol, erow, A_val.astype(jnp.float32), wptr_pad)
    return out[:_N]
